# baseline (speedup 1.0000x reference)
import functools
import jax
import jax.numpy as jnp
from jax import lax
from jax.experimental import pallas as pl
from jax.experimental.pallas import tpu as pltpu
from jax.experimental.pallas import tpu_sc as plsc

B = 8
N = 20000
TOP_K = 750
IOU_THRESH = 0.01
CONF_THRESH = 0.3
V0, V1 = 0.1, 0.2

CH = 400
NF = 10
CHW = NF * CH
NCHUNKS = N // CH
VPC = CH // 16
NBUF = 5
LCAP = N + 16
OUTL = 768
NEG_INF = float("-inf")


def _sc_body(packed_h,
             ox1_h, oy1_h, ox2_h, oy2_h, osc_h, ocnt_h,
             bufs, L1, L2, L3, L4, L5,
             O1, O2, O3, O4, O5, OC, sems, wsem):
    wid = lax.axis_index("s") * 2 + lax.axis_index("c")

    @pl.when(wid < B)
    def _work():
        b = wid
        lane = lax.broadcasted_iota(jnp.int32, (16,), 0)
        zero16 = jnp.zeros((16,), jnp.float32)

        def zbody(i, _):
            O1[pl.ds(i * 16, 16)] = zero16
            O2[pl.ds(i * 16, 16)] = zero16
            O3[pl.ds(i * 16, 16)] = zero16
            O4[pl.ds(i * 16, 16)] = zero16
            O5[pl.ds(i * 16, 16)] = zero16
            return 0
        lax.fori_loop(0, OUTL // 16, zbody, 0)

        base = b * (NCHUNKS * CHW)
        for k in range(NBUF):
            pltpu.async_copy(packed_h.at[pl.ds(base + k * CHW, CHW)],
                             bufs[k], sems[k])

        def process(buf, carry):
            def vec(i, carry2):
                w2, mcv2 = carry2
                def fld(f):
                    return buf[pl.ds(f * CH + i * 16, 16)]
                a0 = fld(0)
                a1 = fld(1)
                m01 = jnp.maximum(a0, a1)
                e0 = jnp.exp(a0 - m01)
                e1 = jnp.exp(a1 - m01)
                sc = e1 / (e0 + e1)

                pxv = fld(6)
                pyv = fld(7)
                pwv = fld(8)
                phv = fld(9)
                cx = pxv + fld(2) * V0 * pwv
                cy = pyv + fld(3) * V0 * phv
                bw = pwv * jnp.exp(fld(4) * V1)
                bh = phv * jnp.exp(fld(5) * V1)
                tx = cx - bw / 2.0
                ty = cy - bh / 2.0
                x1 = tx * 1024.0
                y1 = ty * 1024.0
                x2 = (tx + bw) * 1024.0
                y2 = (ty + bh) * 1024.0

                m = sc > CONF_THRESH
                cmax = jnp.maximum(jnp.maximum(x1, y1), jnp.maximum(x2, y2))
                mcv3 = jnp.maximum(mcv2, jnp.where(m, cmax, NEG_INF))

                dst = pl.ds(w2, 16)
                plsc.store_compressed(L1.at[dst], x1, mask=m)
                plsc.store_compressed(L2.at[dst], y1, mask=m)
                plsc.store_compressed(L3.at[dst], x2, mask=m)
                plsc.store_compressed(L4.at[dst], y2, mask=m)
                plsc.store_compressed(L5.at[dst], sc, mask=m)
                w3 = w2 + jnp.sum(m.astype(jnp.int32))
                return w3, mcv3

            return lax.fori_loop(0, VPC, vec, carry)

        def chunk(i, carry):
            for k in range(NBUF):
                c = i * NBUF + k
                pltpu.make_async_copy(
                    packed_h.at[pl.ds(0, CHW)], bufs[k], sems[k]).wait()
                carry = process(bufs[k], carry)

                @pl.when(i < NCHUNKS // NBUF - 1)
                def _refill():
                    pltpu.async_copy(
                        packed_h.at[pl.ds(base + (c + NBUF) * CHW, CHW)],
                        bufs[k], sems[k])
            return carry

        nlive, mcv = lax.fori_loop(
            0, NCHUNKS // NBUF, chunk,
            (jnp.int32(0), jnp.full((16,), NEG_INF, jnp.float32)))

        mc = jnp.max(mcv)
        finite = (mc == mc) & (jnp.abs(mc) != jnp.inf)
        off = jnp.where(finite, mc, 0.0) + 1.0

        nch0 = (nlive + 15) // 16

        def scanb(i, carry):
            bms, b1, b2, b3, b4, bpos = carry
            p = i * 16
            sl = pl.ds(p, 16)
            x1v = L1[sl] + off
            y1v = L2[sl] + off
            x2v = L3[sl] + off
            y2v = L4[sl] + off
            L1[sl] = x1v
            L2[sl] = y1v
            L3[sl] = x2v
            L4[sl] = y2v
            scv = L5[sl]
            vmask = (p + lane) < nlive
            smk = jnp.where(vmask, scv, NEG_INF)
            chm = jnp.max(smk)
            is_new = chm > bms

            def newf(args):
                x1v, y1v, x2v, y2v = args
                ln = jnp.min(jnp.where(smk == chm, lane, 16))
                selm = lane == ln
                ex = lambda v: jnp.max(jnp.where(selm, v, NEG_INF))
                return (chm, ex(x1v), ex(y1v), ex(x2v), ex(y2v), p + ln)

            return lax.cond(
                is_new, newf,
                lambda args: (bms, b1, b2, b3, b4, bpos),
                (x1v, y1v, x2v, y2v))

        best0 = lax.fori_loop(
            0, nch0, scanb,
            (jnp.float32(NEG_INF), jnp.float32(0), jnp.float32(0),
             jnp.float32(0), jnp.float32(0), jnp.int32(0)))

        def cond(st):
            cnt, nlive_, bms, b1, b2, b3, b4, bpos = st
            return (bms > NEG_INF) & (cnt < TOP_K)

        def body(st):
            cnt, nlive_, bms, b1, b2, b3, b4, bpos = st
            cm = lane == 0
            iv = jnp.full((16,), cnt, jnp.int32)
            plsc.store_scatter(O1, [iv], jnp.full((16,), b1 - off), mask=cm)
            plsc.store_scatter(O2, [iv], jnp.full((16,), b2 - off), mask=cm)
            plsc.store_scatter(O3, [iv], jnp.full((16,), b3 - off), mask=cm)
            plsc.store_scatter(O4, [iv], jnp.full((16,), b4 - off), mask=cm)
            plsc.store_scatter(O5, [iv], jnp.full((16,), bms), mask=cm)
            cnt2 = cnt + 1

            karea = (b3 - b1) * (b4 - b2)
            nch = (nlive_ + 31) // 32

            def sub(p, carry):
                w, nbms, n1, n2, n3, n4, npos = carry
                sl = pl.ds(p, 16)
                x1v = L1[sl]
                y1v = L2[sl]
                x2v = L3[sl]
                y2v = L4[sl]
                scv = L5[sl]
                vmask = (p + lane) < nlive_
                areav = (x2v - x1v) * (y2v - y1v)
                xx1 = jnp.maximum(b1, x1v)
                yy1 = jnp.maximum(b2, y1v)
                xx2 = jnp.minimum(b3, x2v)
                yy2 = jnp.minimum(b4, y2v)
                iw = jnp.maximum(xx2 - xx1, 0.0)
                ih = jnp.maximum(yy2 - yy1, 0.0)
                inter = iw * ih
                iou = inter / (karea + areav - inter)
                keepm = (~(iou > IOU_THRESH)) & vmask & ((p + lane) != bpos)

                dst = pl.ds(w, 16)
                plsc.store_compressed(L1.at[dst], x1v, mask=keepm)
                plsc.store_compressed(L2.at[dst], y1v, mask=keepm)
                plsc.store_compressed(L3.at[dst], x2v, mask=keepm)
                plsc.store_compressed(L4.at[dst], y2v, mask=keepm)
                plsc.store_compressed(L5.at[dst], scv, mask=keepm)
                nk = jnp.sum(keepm.astype(jnp.int32))

                smk = jnp.where(keepm, scv, NEG_INF)
                chm = jnp.max(smk)
                is_new = chm > nbms

                def newf(args):
                    x1v, y1v, x2v, y2v, keepm = args
                    ln = jnp.min(jnp.where(smk == chm, lane, 16))
                    selm = lane == ln
                    ex = lambda v: jnp.max(jnp.where(selm, v, NEG_INF))
                    rank = jnp.sum((keepm & (lane < ln)).astype(jnp.int32))
                    return (chm, ex(x1v), ex(y1v), ex(x2v), ex(y2v), w + rank)

                nbms, n1, n2, n3, n4, npos = lax.cond(
                    is_new, newf,
                    lambda args: (nbms, n1, n2, n3, n4, npos),
                    (x1v, y1v, x2v, y2v, keepm))
                return w + nk, nbms, n1, n2, n3, n4, npos

            def passb(i, carry):
                carry = sub(i * 32, carry)
                carry = sub(i * 32 + 16, carry)
                return carry

            res = lax.fori_loop(
                0, nch, passb,
                (jnp.int32(0), jnp.float32(NEG_INF), jnp.float32(0),
                 jnp.float32(0), jnp.float32(0), jnp.float32(0),
                 jnp.int32(0)))
            w, nbms, n1, n2, n3, n4, npos = res
            return cnt2, w, nbms, n1, n2, n3, n4, npos

        st = lax.while_loop(
            cond, body,
            (jnp.int32(0), nlive, best0[0], best0[1], best0[2], best0[3],
             best0[4], best0[5]))
        cnt_final = st[0]

        OC[...] = jnp.full((16,), cnt_final, jnp.int32)
        wcps = [
            pltpu.async_copy(O1, ox1_h.at[pl.ds(b * OUTL, OUTL)], wsem),
            pltpu.async_copy(O2, oy1_h.at[pl.ds(b * OUTL, OUTL)], wsem),
            pltpu.async_copy(O3, ox2_h.at[pl.ds(b * OUTL, OUTL)], wsem),
            pltpu.async_copy(O4, oy2_h.at[pl.ds(b * OUTL, OUTL)], wsem),
            pltpu.async_copy(O5, osc_h.at[pl.ds(b * OUTL, OUTL)], wsem),
            pltpu.async_copy(OC, ocnt_h.at[pl.ds(b * 16, 16)], wsem),
        ]
        for cp in wcps:
            cp.wait()


_mesh = plsc.VectorSubcoreMesh(
    core_axis_name="c", subcore_axis_name="s", num_cores=2, num_subcores=16)

_sc_call = functools.partial(
    pl.kernel,
    out_type=[jax.ShapeDtypeStruct((B * OUTL,), jnp.float32)] * 5
    + [jax.ShapeDtypeStruct((B * 16,), jnp.int32)],
    mesh=_mesh,
    scratch_types=[tuple(pltpu.VMEM((CHW,), jnp.float32) for _ in range(NBUF))]
    + [pltpu.VMEM((LCAP,), jnp.float32)] * 5
    + [pltpu.VMEM((OUTL,), jnp.float32)] * 5
    + [pltpu.VMEM((16,), jnp.int32),
       tuple(pltpu.SemaphoreType.DMA for _ in range(NBUF)),
       pltpu.SemaphoreType.DMA],
    compiler_params=pltpu.CompilerParams(needs_layout_passes=False),
)(_sc_body)


@jax.jit
def kernel(boxes_logits, cls_logits, priors):
    l0 = cls_logits[:, :, 0]
    l1 = cls_logits[:, :, 1]
    lx = boxes_logits[:, :, 0]
    ly = boxes_logits[:, :, 1]
    lw = boxes_logits[:, :, 2]
    lh = boxes_logits[:, :, 3]
    pr = jnp.broadcast_to(priors.T[None], (B, 4, N))
    fields = jnp.stack([l0, l1, lx, ly, lw, lh], axis=1)
    fields = jnp.concatenate([fields, pr], axis=1)
    packed = fields.reshape(B, NF, NCHUNKS, CH)
    packed = packed.transpose(0, 2, 1, 3).reshape(-1)

    ox1, oy1, ox2, oy2, osc, ocnt = _sc_call(packed)

    ox1 = ox1.reshape(B, OUTL)
    oy1 = oy1.reshape(B, OUTL)
    ox2 = ox2.reshape(B, OUTL)
    oy2 = oy2.reshape(B, OUTL)
    osc = osc.reshape(B, OUTL)
    pred_boxes = jnp.stack(
        [ox1[:, :TOP_K], oy1[:, :TOP_K], ox2[:, :TOP_K], oy2[:, :TOP_K]],
        axis=-1,
    )
    pred_scores = osc[:, :TOP_K]
    counts = ocnt.reshape(B, 16)[:, 0]
    slot = jnp.arange(TOP_K, dtype=jnp.int32)[None, :]
    pred_labels = jnp.where(slot < counts[:, None], 1, 0).astype(jnp.int64)
    return pred_boxes, pred_scores, pred_labels, counts
